# baseline (speedup 1.0000x reference)
import functools

import jax
import jax.numpy as jnp
from jax import lax
from jax.experimental import pallas as pl
from jax.experimental.pallas import tpu as pltpu
from jax.experimental.pallas import tpu_sc as plsc

_BB = 4
_NB = 16
_P = 1024
_C = 256
_N = 1024
_CPW = 8


def _argmin_body(zb_ref, emb_ref, idx_ref, et_ref):
    emb = emb_ref[...]
    e_sq = jnp.sum(emb * emb, axis=1, keepdims=True)

    @pl.when(pl.program_id(0) == 0)
    def _build_et():
        ey = (lax.broadcasted_iota(jnp.int32, (_C, _C), 0)
              == lax.broadcasted_iota(jnp.int32, (_C, _C), 1)
              ).astype(jnp.bfloat16)
        et_ref[...] = lax.dot_general(
            ey, emb.astype(jnp.bfloat16), (((1,), (1,)), ((), ())),
            preferred_element_type=jnp.float32)

    for j in range(_BB):
        zb = zb_ref[j]
        s = lax.dot_general(emb, zb, (((1,), (0,)), ((), ())),
                            preferred_element_type=jnp.float32)
        dist = e_sq - 2.0 * s
        idx_ref[j, 0, :] = jnp.argmin(dist, axis=0)


def _tc_argmin(zr, embedding):
    b, c, p = zr.shape
    n, d = embedding.shape
    return pl.pallas_call(
        _argmin_body,
        grid=(b // _BB,),
        in_specs=[
            pl.BlockSpec((_BB, c, p), lambda i: (i, 0, 0)),
            pl.BlockSpec((n, d), lambda i: (0, 0)),
        ],
        out_specs=[
            pl.BlockSpec((_BB, 1, p), lambda i: (i, 0, 0)),
            pl.BlockSpec((d, n), lambda i: (0, 0)),
        ],
        out_shape=[
            jax.ShapeDtypeStruct((b, 1, p), jnp.int32),
            jax.ShapeDtypeStruct((d, n), jnp.float32),
        ],
    )(zr, embedding)


def _sc_gather_body(et_hbm, idx_hbm, out_hbm, tbl_v, idx_v, ob0, ob1,
                    sem0, sem1):
    wid = lax.axis_index("s") * 2 + lax.axis_index("c")
    c0 = wid * _CPW
    pltpu.sync_copy(et_hbm.at[pl.ds(c0 * _N, _CPW * _N)], tbl_v)
    pltpu.sync_copy(idx_hbm, idx_v)

    obs = (ob0, ob1)
    sems = (sem0, sem1)
    copies = [None, None]
    for bi in range(_NB):
        j = bi & 1
        ob = obs[j]
        if copies[j] is not None:
            copies[j].wait()

        def chunk_body(k, _, bi=bi, ob=ob):
            idxv = idx_v[pl.ds(bi * _P + k * 16, 16)]
            for cc in range(_CPW):
                vals = plsc.load_gather(tbl_v, [idxv + (cc * _N)])
                ob[pl.ds(cc * _P + k * 16, 16)] = vals
            return 0

        lax.fori_loop(0, _P // 16, chunk_body, 0, unroll=8)
        cp = pltpu.make_async_copy(
            ob, out_hbm.at[pl.ds(bi * (_C * _P) + c0 * _P, _CPW * _P)],
            sems[j])
        cp.start()
        copies[j] = cp
    copies[0].wait()
    copies[1].wait()


def _sc_gather(et_flat, idx_flat):
    mesh = plsc.VectorSubcoreMesh(core_axis_name="c", subcore_axis_name="s")
    f = functools.partial(
        pl.kernel,
        mesh=mesh,
        out_type=jax.ShapeDtypeStruct((_NB * _C * _P,), jnp.float32),
        scratch_types=[
            pltpu.VMEM((_CPW * _N,), jnp.float32),
            pltpu.VMEM((_NB * _P,), jnp.int32),
            pltpu.VMEM((_CPW * _P,), jnp.float32),
            pltpu.VMEM((_CPW * _P,), jnp.float32),
            pltpu.SemaphoreType.DMA,
            pltpu.SemaphoreType.DMA,
        ],
        compiler_params=pltpu.CompilerParams(needs_layout_passes=False),
    )(_sc_gather_body)
    return f(et_flat, idx_flat)


def kernel(z, embedding):
    b, c, h, w = z.shape
    n, d = embedding.shape
    p = h * w
    zr = z.reshape(b, c, p)
    idx3, et = _tc_argmin(zr, embedding)
    qflat = _sc_gather(et.reshape(d * n), idx3.reshape(b * p))
    return (qflat.reshape(b, c, h, w), 0.0, idx3.reshape(b, p))


# speedup vs baseline: 1.2207x; 1.2207x over previous
import functools

import jax
import jax.numpy as jnp
from jax import lax
from jax.experimental import pallas as pl
from jax.experimental.pallas import tpu as pltpu
from jax.experimental.pallas import tpu_sc as plsc

_BB = 4
_NB = 16
_P = 1024
_C = 256
_N = 1024
_CPW = 8


def _argmin_body(zb_ref, emb_ref, idx_ref, et_ref):
    emb = emb_ref[...]
    e_sq = jnp.sum(emb * emb, axis=1, keepdims=True)

    @pl.when(pl.program_id(0) == 0)
    def _build_et():
        ey = (lax.broadcasted_iota(jnp.int32, (_C, _C), 0)
              == lax.broadcasted_iota(jnp.int32, (_C, _C), 1)
              ).astype(jnp.bfloat16)
        et_ref[...] = lax.dot_general(
            ey, emb.astype(jnp.bfloat16), (((1,), (1,)), ((), ())),
            preferred_element_type=jnp.float32)

    for j in range(_BB):
        zb = zb_ref[j]
        s = lax.dot_general(emb, zb, (((1,), (0,)), ((), ())),
                            preferred_element_type=jnp.float32)
        dist = e_sq - 2.0 * s
        idx_ref[j, 0, :] = jnp.argmin(dist, axis=0)


def _tc_argmin(zr, embedding):
    b, c, p = zr.shape
    n, d = embedding.shape
    return pl.pallas_call(
        _argmin_body,
        grid=(b // _BB,),
        in_specs=[
            pl.BlockSpec((_BB, c, p), lambda i: (i, 0, 0)),
            pl.BlockSpec((n, d), lambda i: (0, 0)),
        ],
        out_specs=[
            pl.BlockSpec((_BB, 1, p), lambda i: (i, 0, 0)),
            pl.BlockSpec((d, n), lambda i: (0, 0)),
        ],
        out_shape=[
            jax.ShapeDtypeStruct((b, 1, p), jnp.int32),
            jax.ShapeDtypeStruct((d, n), jnp.float32),
        ],
    )(zr, embedding)


def _sc_gather_body(et_hbm, idx_hbm, out_hbm, tbl_v, idx_v, ob0, ob1,
                    sem0, sem1):
    wid = lax.axis_index("s") * 2 + lax.axis_index("c")
    c0 = wid * _CPW
    pltpu.sync_copy(et_hbm.at[pl.ds(c0 * _N, _CPW * _N)], tbl_v)
    pltpu.sync_copy(idx_hbm, idx_v)

    obs = (ob0, ob1)
    sems = (sem0, sem1)
    copies = [None, None]
    for bi in range(_NB):
        j = bi & 1
        ob = obs[j]
        if copies[j] is not None:
            copies[j].wait()

        @plsc.parallel_loop(0, _P // 16, unroll=8)
        def chunk_body(k, bi=bi, ob=ob):
            idxv = idx_v[pl.ds(bi * _P + k * 16, 16)]
            for cc in range(_CPW):
                vals = plsc.load_gather(tbl_v, [idxv + (cc * _N)])
                ob[pl.ds(cc * _P + k * 16, 16)] = vals
        cp = pltpu.make_async_copy(
            ob, out_hbm.at[pl.ds(bi * (_C * _P) + c0 * _P, _CPW * _P)],
            sems[j])
        cp.start()
        copies[j] = cp
    copies[0].wait()
    copies[1].wait()


def _sc_gather(et_flat, idx_flat):
    mesh = plsc.VectorSubcoreMesh(core_axis_name="c", subcore_axis_name="s")
    f = functools.partial(
        pl.kernel,
        mesh=mesh,
        out_type=jax.ShapeDtypeStruct((_NB * _C * _P,), jnp.float32),
        scratch_types=[
            pltpu.VMEM((_CPW * _N,), jnp.float32),
            pltpu.VMEM((_NB * _P,), jnp.int32),
            pltpu.VMEM((_CPW * _P,), jnp.float32),
            pltpu.VMEM((_CPW * _P,), jnp.float32),
            pltpu.SemaphoreType.DMA,
            pltpu.SemaphoreType.DMA,
        ],
        compiler_params=pltpu.CompilerParams(needs_layout_passes=False),
    )(_sc_gather_body)
    return f(et_flat, idx_flat)


def kernel(z, embedding):
    b, c, h, w = z.shape
    n, d = embedding.shape
    p = h * w
    zr = z.reshape(b, c, p)
    idx3, et = _tc_argmin(zr, embedding)
    qflat = _sc_gather(et.reshape(d * n), idx3.reshape(b * p))
    return (qflat.reshape(b, c, h, w), 0.0, idx3.reshape(b, p))


# speedup vs baseline: 2.6311x; 2.1553x over previous
import jax
import jax.numpy as jnp
from jax import lax
from jax.experimental import pallas as pl

_BB = 2


def _vq_body(zb_ref, emb_ref, q_ref, idx_ref):
    emb = emb_ref[...]
    n, d = emb.shape
    p = zb_ref.shape[-1]
    e_sq = jnp.sum(emb * emb, axis=1, keepdims=True)
    emb_bf = emb.astype(jnp.bfloat16)
    iota0 = lax.broadcasted_iota(jnp.int32, (n, p), 0)
    for j in range(_BB):
        zb = zb_ref[j]
        s = lax.dot_general(emb, zb, (((1,), (0,)), ((), ())),
                            preferred_element_type=jnp.float32)
        dist = e_sq - 2.0 * s
        idx = jnp.argmin(dist, axis=0)
        idx_ref[j, 0, :] = idx
        onehot = (iota0 == idx[None, :]).astype(jnp.bfloat16)
        q = lax.dot_general(emb_bf, onehot, (((0,), (0,)), ((), ())),
                            preferred_element_type=jnp.float32)
        q_ref[j] = q


def kernel(z, embedding):
    b, c, h, w = z.shape
    n, d = embedding.shape
    p = h * w
    zr = z.reshape(b, c, p)
    q, idx = pl.pallas_call(
        _vq_body,
        grid=(b // _BB,),
        in_specs=[
            pl.BlockSpec((_BB, c, p), lambda i: (i, 0, 0)),
            pl.BlockSpec((n, d), lambda i: (0, 0)),
        ],
        out_specs=[
            pl.BlockSpec((_BB, c, p), lambda i: (i, 0, 0)),
            pl.BlockSpec((_BB, 1, p), lambda i: (i, 0, 0)),
        ],
        out_shape=[
            jax.ShapeDtypeStruct((b, c, p), jnp.float32),
            jax.ShapeDtypeStruct((b, 1, p), jnp.int32),
        ],
    )(zr, embedding)
    return (q.reshape(b, c, h, w), 0.0, idx.reshape(b, p))


# speedup vs baseline: 2.7805x; 1.0568x over previous
import jax
import jax.numpy as jnp
from jax import lax
from jax.experimental import pallas as pl

_BB = 2


def _vq_body(zb_ref, emb_ref, q_ref, idx_ref):
    emb = emb_ref[...]
    n, d = emb.shape
    p = zb_ref.shape[-1]
    e_sq = jnp.sum(emb * emb, axis=1, keepdims=True)
    emb_bf = emb.astype(jnp.bfloat16)
    em2 = emb * -2.0
    iota0 = lax.broadcasted_iota(jnp.int32, (n, p), 0)
    for j in range(_BB):
        zb = zb_ref[j]
        s = lax.dot_general(em2, zb, (((1,), (0,)), ((), ())),
                            preferred_element_type=jnp.float32)
        dist = e_sq + s
        idx = jnp.argmin(dist, axis=0)
        idx_ref[j, 0, :] = idx
        onehot = (iota0 == idx[None, :]).astype(jnp.bfloat16)
        q = lax.dot_general(emb_bf, onehot, (((0,), (0,)), ((), ())),
                            preferred_element_type=jnp.float32)
        q_ref[j] = q


def kernel(z, embedding):
    b, c, h, w = z.shape
    n, d = embedding.shape
    p = h * w
    zr = z.reshape(b, c, p)
    q, idx = pl.pallas_call(
        _vq_body,
        grid=(b // _BB,),
        in_specs=[
            pl.BlockSpec((_BB, c, p), lambda i: (i, 0, 0)),
            pl.BlockSpec((n, d), lambda i: (0, 0)),
        ],
        out_specs=[
            pl.BlockSpec((_BB, c, p), lambda i: (i, 0, 0)),
            pl.BlockSpec((_BB, 1, p), lambda i: (i, 0, 0)),
        ],
        out_shape=[
            jax.ShapeDtypeStruct((b, c, p), jnp.float32),
            jax.ShapeDtypeStruct((b, 1, p), jnp.int32),
        ],
    )(zr, embedding)
    return (q.reshape(b, c, h, w), 0.0, idx.reshape(b, p))


# speedup vs baseline: 2.7998x; 1.0069x over previous
import jax
import jax.numpy as jnp
from jax import lax
from jax.experimental import pallas as pl

_BB = 4


def _vq_body(zb_ref, emb_ref, q_ref, idx_ref):
    emb = emb_ref[...]
    n, d = emb.shape
    p = zb_ref.shape[-1]
    e_sq = jnp.sum(emb * emb, axis=1, keepdims=True)
    emb_bf = emb.astype(jnp.bfloat16)
    em2 = emb * -2.0
    iota0 = lax.broadcasted_iota(jnp.int32, (n, p), 0)
    for j in range(_BB):
        zb = zb_ref[j]
        s = lax.dot_general(em2, zb, (((1,), (0,)), ((), ())),
                            preferred_element_type=jnp.float32)
        dist = e_sq + s
        idx = jnp.argmin(dist, axis=0)
        idx_ref[j, 0, :] = idx
        onehot = (iota0 == idx[None, :]).astype(jnp.bfloat16)
        q = lax.dot_general(emb_bf, onehot, (((0,), (0,)), ((), ())),
                            preferred_element_type=jnp.float32)
        q_ref[j] = q


def kernel(z, embedding):
    b, c, h, w = z.shape
    n, d = embedding.shape
    p = h * w
    zr = z.reshape(b, c, p)
    q, idx = pl.pallas_call(
        _vq_body,
        grid=(b // _BB,),
        in_specs=[
            pl.BlockSpec((_BB, c, p), lambda i: (i, 0, 0)),
            pl.BlockSpec((n, d), lambda i: (0, 0)),
        ],
        out_specs=[
            pl.BlockSpec((_BB, c, p), lambda i: (i, 0, 0)),
            pl.BlockSpec((_BB, 1, p), lambda i: (i, 0, 0)),
        ],
        out_shape=[
            jax.ShapeDtypeStruct((b, c, p), jnp.float32),
            jax.ShapeDtypeStruct((b, 1, p), jnp.int32),
        ],
    )(zr, embedding)
    return (q.reshape(b, c, h, w), 0.0, idx.reshape(b, p))
